# Initial kernel scaffold; baseline (speedup 1.0000x reference)
import jax
import jax.numpy as jnp
from jax.experimental import pallas as pl


def kernel(x, sym_table, pos_table):
    raise NotImplementedError("write your pallas kernel here")


# speedup vs baseline: 5.5403x; 5.5403x over previous
import functools

import jax
import jax.numpy as jnp
from jax import lax
from jax.experimental import pallas as pl
from jax.experimental.pallas import tpu as pltpu
from jax.experimental.pallas import tpu_sc as plsc

SYM_LEN = 100000
MAX_SEQ_LEN = 200
EMB_DIM = 128
BATCH = 1024
SEQ = 200

_HALF = SEQ // 2
_NW = 32
_CHUNKS_PER_W = BATCH // _NW
_LANES = 16
_VPR = EMB_DIM // _LANES


def _emb_body(x_hbm, sym_hbm, pos_hbm, out_hbm,
              pos_v, idx_v, buf_v, gsem):
    nc = 2
    wid = lax.axis_index("s") * nc + lax.axis_index("c")
    c0 = wid * _CHUNKS_PER_W

    pltpu.sync_copy(pos_hbm, pos_v)

    def start_gather(slot, b):
        pltpu.sync_copy(x_hbm.at[b], idx_v.at[slot])
        for h in range(2):
            pltpu.async_copy(
                sym_hbm.at[idx_v.at[slot, h]],
                buf_v.at[slot, pl.ds(h * _HALF, _HALF)],
                gsem.at[slot],
            )

    def wait_gather(slot, b):
        for h in range(2):
            pltpu.make_async_copy(
                sym_hbm.at[idx_v.at[slot, h]],
                buf_v.at[slot, pl.ds(h * _HALF, _HALF)],
                gsem.at[slot],
            ).wait()

    def add_pos(slot):
        def row(i, _):
            for j in range(_VPR):
                v = pos_v[i, pl.ds(j * _LANES, _LANES)]
                plsc.addupdate(buf_v.at[slot, i, pl.ds(j * _LANES, _LANES)], v)
            return 0

        lax.fori_loop(0, SEQ, row, 0, unroll=False)

    def process(slot, b, last):
        wait_gather(slot, b)
        add_pos(slot)
        pltpu.sync_copy(buf_v.at[slot], out_hbm.at[b])

        @pl.when(jnp.logical_not(last))
        def _():
            start_gather(slot, b + 2)

    start_gather(0, c0)
    start_gather(1, c0 + 1)

    def pair(t, _):
        b0 = c0 + 2 * t
        last = t >= (_CHUNKS_PER_W // 2 - 1)
        process(0, b0, last)
        process(1, b0 + 1, last)
        return 0

    lax.fori_loop(0, _CHUNKS_PER_W // 2, pair, 0, unroll=False)


@jax.jit
def _emb_call(x3, sym_table, pos_table):
    mesh = plsc.VectorSubcoreMesh(core_axis_name="c", subcore_axis_name="s")
    k = functools.partial(
        pl.kernel,
        out_type=jax.ShapeDtypeStruct((BATCH, SEQ, EMB_DIM), jnp.float32),
        mesh=mesh,
        scratch_types=[
            pltpu.VMEM((MAX_SEQ_LEN, EMB_DIM), jnp.float32),
            pltpu.VMEM((2, 2, _HALF), jnp.int32),
            pltpu.VMEM((2, SEQ, EMB_DIM), jnp.float32),
            pltpu.SemaphoreType.DMA((2,)),
        ],
    )(_emb_body)
    return k(x3, sym_table, pos_table)


def kernel(x, sym_table, pos_table):
    x3 = x.astype(jnp.int32).reshape(BATCH, 2, _HALF)
    return _emb_call(x3, sym_table, pos_table)
